# baseline (speedup 1.0000x reference)
import functools

import jax
import jax.numpy as jnp
from jax.experimental import pallas as pl
from jax.experimental.pallas import tpu as pltpu

NJ = 64
H = 100
K = 16
E = H * K
NEDGE = NJ * E
EPS = 1e-5
BIG = 1e30

_f32 = jnp.float32


def _jet_spec(*tail):
    return pl.BlockSpec((1,) + tail, lambda b: (b, 0, 0))


def _w_spec(shape):
    return pl.BlockSpec(shape, lambda b: (0,) * len(shape))


def _iota(shape, dim, dtype=jnp.int32):
    return jax.lax.broadcasted_iota(dtype, shape, dim)


def _dotT(a, b):
    return jax.lax.dot_general(a, b, (((1,), (1,)), ((), ())),
                               preferred_element_type=_f32,
                               precision=jax.lax.Precision.HIGHEST)


def _dot(a, b):
    return jnp.dot(a, b, preferred_element_type=_f32,
                   precision=jax.lax.Precision.HIGHEST)


def _build_rep():
    return (_iota((E, H), 0) // K == _iota((E, H), 1)).astype(_f32)


def _build_g(nbrf):
    rep = _build_rep()
    repn = _dot(rep, nbrf)
    ksel = (_iota((E, K), 1) == _iota((E, K), 0) % K).astype(_f32)
    nbr_val = jnp.sum(repn * ksel, axis=1, keepdims=True)
    g = (jnp.abs(_iota((E, H), 1).astype(_f32) - nbr_val) < 0.5).astype(_f32)
    return rep, g


def _bn_consts(st_ref, g_ref, be_ref):
    mean = st_ref[0:1, :] * (1.0 / NEDGE)
    var = st_ref[1:2, :] * (1.0 / NEDGE) - mean * mean
    s = g_ref[...] * jax.lax.rsqrt(var + EPS)
    t = be_ref[...] - s * mean
    return s, t


def _acc_stats(st_ref, h, b):
    @pl.when(b == 0)
    def _():
        st_ref[...] = jnp.zeros_like(st_ref)
    st_ref[0:1, :] = st_ref[0:1, :] + jnp.sum(h, axis=0, keepdims=True)
    st_ref[1:2, :] = st_ref[1:2, :] + jnp.sum(h * h, axis=0, keepdims=True)


def _run_prep(feats, pos_list, wc_list, wb_list, b1, L):
    nfeat, npos = len(feats), len(pos_list)

    def body(*refs):
        i = 0
        frefs = refs[i:i + nfeat]; i += nfeat
        prefs = refs[i:i + npos]; i += npos
        wcrefs = refs[i:i + nfeat]; i += nfeat
        wbrefs = refs[i:i + nfeat]; i += nfeat
        b1_ref = refs[i]; i += 1
        a_out, bv_out, nbr_out, st_out = refs[i:i + 4]
        b = pl.program_id(0)

        fs = [r[0] for r in frefs]
        ps = [r[0] for r in prefs] if npos else fs

        d_gram = None
        for p in ps:
            d = _dotT(p, p)
            d_gram = d if d_gram is None else d_gram + d
        eye = _iota((H, H), 0) == _iota((H, H), 1)
        dm = jnp.where(eye, d_gram, 0.0)
        rdiag = jnp.sum(dm, axis=1, keepdims=True)
        cdiag = jnp.sum(dm, axis=0, keepdims=True)
        d2 = rdiag + cdiag - 2.0 * d_gram
        d2 = jnp.where(eye, BIG, d2)

        li = _iota((H, H), 1).astype(_f32)
        kl = _iota((H, K), 1)

        def step(k, carry):
            d2c, acc = carry
            m = jnp.min(d2c, axis=1, keepdims=True)
            am = jnp.min(jnp.where(d2c <= m, li, 1e9), axis=1, keepdims=True)
            d2c = jnp.where(jnp.abs(li - am) < 0.5, BIG, d2c)
            acc = jnp.where(kl == k, am, acc)
            return d2c, acc

        _, nbrf = jax.lax.fori_loop(
            0, K, step, (d2, jnp.zeros((H, K), _f32)))

        a = b1_ref[...]
        bv = None
        for f, wc, wb in zip(fs, wcrefs, wbrefs):
            a = a + _dot(f, wc[...])
            pb = _dot(f, wb[...])
            bv = pb if bv is None else bv + pb

        rep, g = _build_g(nbrf)
        h1 = _dot(rep, a) + _dot(g, bv)

        a_out[0] = a
        bv_out[0] = bv
        nbr_out[0] = nbrf
        _acc_stats(st_out, h1, b)

    in_specs = (
        [_jet_spec(H, f.shape[-1]) for f in feats]
        + [_jet_spec(H, p.shape[-1]) for p in pos_list]
        + [_w_spec(w.shape) for w in wc_list]
        + [_w_spec(w.shape) for w in wb_list]
        + [_w_spec(b1.shape)]
    )
    out_shape = [
        jax.ShapeDtypeStruct((NJ, H, L), _f32),
        jax.ShapeDtypeStruct((NJ, H, L), _f32),
        jax.ShapeDtypeStruct((NJ, H, K), _f32),
        jax.ShapeDtypeStruct((8, L), _f32),
    ]
    out_specs = [_jet_spec(H, L), _jet_spec(H, L), _jet_spec(H, K),
                 _w_spec((8, L))]
    return pl.pallas_call(
        body, grid=(NJ,), in_specs=in_specs, out_specs=out_specs,
        out_shape=out_shape,
    )(*feats, *pos_list, *wc_list, *wb_list, b1)


def _run_mid(a, bv, nbrf, st1, gamma, beta, w2, b2, L):
    def body(a_ref, bv_ref, nbr_ref, st_ref, g_ref, be_ref, w_ref, b2_ref,
             h2_out, st2_out):
        b = pl.program_id(0)
        s, t = _bn_consts(st_ref, g_ref, be_ref)
        rep, g = _build_g(nbr_ref[0])
        m1 = jnp.maximum(_dot(rep, a_ref[0] * s) + _dot(g, bv_ref[0] * s) + t,
                         0.0)
        h2 = _dot(m1, w_ref[...]) + b2_ref[...]
        h2_out[0] = h2
        _acc_stats(st2_out, h2, b)

    return pl.pallas_call(
        body, grid=(NJ,),
        in_specs=[_jet_spec(H, L), _jet_spec(H, L), _jet_spec(H, K),
                  _w_spec((8, L)), _w_spec((1, L)), _w_spec((1, L)),
                  _w_spec((L, L)), _w_spec((1, L))],
        out_specs=[_jet_spec(E, L), _w_spec((8, L))],
        out_shape=[jax.ShapeDtypeStruct((NJ, E, L), _f32),
                   jax.ShapeDtypeStruct((8, L), _f32)],
    )(a, bv, nbrf, st1, gamma, beta, w2, b2)


def _run_mid2(h2, st2, gamma, beta, w3, b3, L):
    def body(h_ref, st_ref, g_ref, be_ref, w_ref, b3_ref, h3_out, st3_out):
        b = pl.program_id(0)
        s, t = _bn_consts(st_ref, g_ref, be_ref)
        m2 = jnp.maximum(s * h_ref[0] + t, 0.0)
        h3 = _dot(m2, w_ref[...]) + b3_ref[...]
        h3_out[0] = h3
        _acc_stats(st3_out, h3, b)

    return pl.pallas_call(
        body, grid=(NJ,),
        in_specs=[_jet_spec(E, L), _w_spec((8, L)), _w_spec((1, L)),
                  _w_spec((1, L)), _w_spec((L, L)), _w_spec((1, L))],
        out_specs=[_jet_spec(E, L), _w_spec((8, L))],
        out_shape=[jax.ShapeDtypeStruct((NJ, E, L), _f32),
                   jax.ShapeDtypeStruct((8, L), _f32)],
    )(h2, st2, gamma, beta, w3, b3)


def _rept():
    return (_iota((H, E), 1) // K == _iota((H, E), 0)).astype(_f32)


def _run_agg(h3, st3, gamma, beta, L):
    def body(h_ref, st_ref, g_ref, be_ref, hn_out):
        s, t = _bn_consts(st_ref, g_ref, be_ref)
        m3 = jnp.maximum(s * h_ref[0] + t, 0.0)
        hn_out[0] = _dot(_rept(), m3) * (1.0 / K)

    return pl.pallas_call(
        body, grid=(NJ,),
        in_specs=[_jet_spec(E, L), _w_spec((8, L)), _w_spec((1, L)),
                  _w_spec((1, L))],
        out_specs=_jet_spec(H, L),
        out_shape=jax.ShapeDtypeStruct((NJ, H, L), _f32),
    )(h3, st3, gamma, beta)


def _run_final(h3, st3, gamma, beta, prev_feats, w1_parts, b1f, w2f, b2f, L):
    nprev = len(prev_feats)

    def body(*refs):
        i = 0
        h_ref, st_ref, g_ref, be_ref = refs[i:i + 4]; i += 4
        prefs = refs[i:i + nprev]; i += nprev
        w1refs = refs[i:i + 1 + nprev]; i += 1 + nprev
        b1_ref, w2_ref, b2_ref = refs[i:i + 3]; i += 3
        out_ref = refs[i]; i += 1
        yacc = refs[i]
        j = pl.program_id(0)

        s, t = _bn_consts(st_ref, g_ref, be_ref)
        m3 = jnp.maximum(s * h_ref[0] + t, 0.0)
        hn = _dot(_rept(), m3) * (1.0 / K)

        y = jnp.sum(hn, axis=0, keepdims=True) * (1.0 / H)
        y = _dot(y, w1refs[0][...]) + b1_ref[...]
        for pr, wr in zip(prefs, w1refs[1:]):
            pm = jnp.sum(pr[0], axis=0, keepdims=True) * (1.0 / H)
            y = y + _dot(pm, wr[...])
        yacc[pl.ds(j, 1), :] = y

        @pl.when(j == NJ - 1)
        def _():
            out_ref[...] = _dot(yacc[...], w2_ref[...]) + b2_ref[...]

    in_specs = (
        [_jet_spec(E, L), _w_spec((8, L)), _w_spec((1, L)), _w_spec((1, L))]
        + [_jet_spec(H, f.shape[-1]) for f in prev_feats]
        + [_w_spec(w.shape) for w in w1_parts]
        + [_w_spec(b1f.shape), _w_spec(w2f.shape), _w_spec(b2f.shape)]
    )
    return pl.pallas_call(
        body, grid=(NJ,),
        in_specs=in_specs,
        out_specs=pl.BlockSpec((NJ, 5), lambda b: (0, 0)),
        out_shape=jax.ShapeDtypeStruct((NJ, 5), _f32),
        scratch_shapes=[pltpu.VMEM((NJ, 256), _f32)],
    )(h3, st3, gamma, beta, *prev_feats, *w1_parts, b1f, w2f, b2f)


def _conv_weights(layers, ins, piece_dims):
    w1 = layers[0]["W"]
    wa, wb = w1[:ins], w1[ins:]
    wc = wa - wb
    offs = []
    o = 0
    for d in piece_dims:
        offs.append((o, o + d))
        o += d
    wc_list = [wc[a:b] for a, b in offs]
    wb_list = [wb[a:b] for a, b in offs]

    def row(v):
        return v.reshape(1, -1)

    return {
        "wc": wc_list, "wb": wb_list, "b1": row(layers[0]["b"]),
        "g1": row(layers[0]["gamma"]), "be1": row(layers[0]["beta"]),
        "w2": layers[1]["W"], "b2": row(layers[1]["b"]),
        "g2": row(layers[1]["gamma"]), "be2": row(layers[1]["beta"]),
        "w3": layers[2]["W"], "b3": row(layers[2]["b"]),
        "g3": row(layers[2]["gamma"]), "be3": row(layers[2]["beta"]),
    }


def _conv_block(feats, pos_list, layers, ins, L):
    cw = _conv_weights(layers, ins, [f.shape[-1] for f in feats])
    a, bv, nbrf, st1 = _run_prep(feats, pos_list, cw["wc"], cw["wb"],
                                 cw["b1"], L)
    h2, st2 = _run_mid(a, bv, nbrf, st1, cw["g1"], cw["be1"], cw["w2"],
                       cw["b2"], L)
    h3, st3 = _run_mid2(h2, st2, cw["g2"], cw["be2"], cw["w3"], cw["b3"], L)
    return h3, st3, cw


def kernel(x, params):
    x = x.astype(_f32)

    h3, st3, cw = _conv_block([x], [x[:, :, :2]], params["conv0"], 4, 64)
    h0 = _run_agg(h3, st3, cw["g3"], cw["be3"], 64)

    feats1 = [h0, x]
    h3, st3, cw = _conv_block(feats1, [], params["conv1"], 68, 128)
    h1n = _run_agg(h3, st3, cw["g3"], cw["be3"], 128)

    feats2 = [h1n, h0, x]
    h3, st3, cw = _conv_block(feats2, [], params["conv2"], 196, 256)

    w1 = params["fc1"]["W"]
    w1_parts = [w1[0:256], w1[256:384], w1[384:448], w1[448:452]]
    out = _run_final(h3, st3, cw["g3"], cw["be3"], feats2,
                     w1_parts, params["fc1"]["b"].reshape(1, -1),
                     params["fc2"]["W"], params["fc2"]["b"].reshape(1, -1),
                     256)
    return out


# speedup vs baseline: 1.1915x; 1.1915x over previous
import functools

import jax
import jax.numpy as jnp
from jax.experimental import pallas as pl
from jax.experimental.pallas import tpu as pltpu

NJ = 64
H = 100
K = 16
E = H * K
NEDGE = NJ * E
EPS = 1e-5
BIG = 1e30

_f32 = jnp.float32


def _jet_spec(*tail):
    return pl.BlockSpec((1,) + tail, lambda b: (b, 0, 0))


def _w_spec(shape):
    return pl.BlockSpec(shape, lambda b: (0,) * len(shape))


def _iota(shape, dim, dtype=jnp.int32):
    return jax.lax.broadcasted_iota(dtype, shape, dim)


def _dotT(a, b):
    return jax.lax.dot_general(a, b, (((1,), (1,)), ((), ())),
                               preferred_element_type=_f32,
                               precision=jax.lax.Precision.HIGHEST)


def _dot(a, b):
    return jnp.dot(a, b, preferred_element_type=_f32,
                   precision=jax.lax.Precision.HIGHEST)


def _build_rep():
    return (_iota((E, H), 0) // K == _iota((E, H), 1)).astype(_f32)


def _build_g(nbrf):
    rep = _build_rep()
    repn = _dot(rep, nbrf)
    ksel = (_iota((E, K), 1) == _iota((E, K), 0) % K).astype(_f32)
    nbr_val = jnp.sum(repn * ksel, axis=1, keepdims=True)
    g = (jnp.abs(_iota((E, H), 1).astype(_f32) - nbr_val) < 0.5).astype(_f32)
    return rep, g


def _bn_consts(st_ref, g_ref, be_ref):
    mean = st_ref[0:1, :] * (1.0 / NEDGE)
    var = st_ref[1:2, :] * (1.0 / NEDGE) - mean * mean
    s = g_ref[...] * jax.lax.rsqrt(var + EPS)
    t = be_ref[...] - s * mean
    return s, t


def _acc_stats(st_ref, h, b):
    @pl.when(b == 0)
    def _():
        st_ref[...] = jnp.zeros_like(st_ref)
    st_ref[0:1, :] = st_ref[0:1, :] + jnp.sum(h, axis=0, keepdims=True)
    st_ref[1:2, :] = st_ref[1:2, :] + jnp.sum(h * h, axis=0, keepdims=True)


def _run_prep(feats, pos_list, wc_list, wb_list, b1, L):
    nfeat, npos = len(feats), len(pos_list)

    def body(*refs):
        i = 0
        frefs = refs[i:i + nfeat]; i += nfeat
        prefs = refs[i:i + npos]; i += npos
        wcrefs = refs[i:i + nfeat]; i += nfeat
        wbrefs = refs[i:i + nfeat]; i += nfeat
        b1_ref = refs[i]; i += 1
        a_out, bv_out, nbr_out, st_out = refs[i:i + 4]
        b = pl.program_id(0)

        fs = [r[0] for r in frefs]
        ps = [r[0] for r in prefs] if npos else fs

        d_gram = None
        for p in ps:
            d = _dotT(p, p)
            d_gram = d if d_gram is None else d_gram + d
        eye = _iota((H, H), 0) == _iota((H, H), 1)
        dm = jnp.where(eye, d_gram, 0.0)
        rdiag = jnp.sum(dm, axis=1, keepdims=True)
        cdiag = jnp.sum(dm, axis=0, keepdims=True)
        d2 = rdiag + cdiag - 2.0 * d_gram
        d2 = jnp.where(eye, BIG, d2)

        li = _iota((H, H), 1).astype(_f32)
        kl = _iota((H, K), 1)

        def step(k, carry):
            d2c, acc, adj = carry
            m = jnp.min(d2c, axis=1, keepdims=True)
            am = jnp.min(jnp.where(d2c <= m, li, 1e9), axis=1, keepdims=True)
            sel = jnp.abs(li - am) < 0.5
            d2c = jnp.where(sel, BIG, d2c)
            adj = jnp.where(sel, 1.0, adj)
            acc = jnp.where(kl == k, am, acc)
            return d2c, acc, adj

        _, nbrf, adj = jax.lax.fori_loop(
            0, K, step, (d2, jnp.zeros((H, K), _f32),
                         jnp.zeros((H, H), _f32)))

        a = b1_ref[...]
        bv = None
        for f, wc, wb in zip(fs, wcrefs, wbrefs):
            a = a + _dot(f, wc[...])
            pb = _dot(f, wb[...])
            bv = pb if bv is None else bv + pb

        c_row = jnp.sum(adj, axis=0, keepdims=True)
        s1 = float(K) * jnp.sum(a, axis=0, keepdims=True) + _dot(c_row, bv)
        sq1 = (float(K) * jnp.sum(a * a, axis=0, keepdims=True)
               + _dot(c_row, bv * bv)
               + 2.0 * jnp.sum(a * _dot(adj, bv), axis=0, keepdims=True))

        a_out[0] = a
        bv_out[0] = bv
        nbr_out[0] = nbrf

        @pl.when(b == 0)
        def _():
            st_out[...] = jnp.zeros_like(st_out)
        st_out[0:1, :] = st_out[0:1, :] + s1
        st_out[1:2, :] = st_out[1:2, :] + sq1

    in_specs = (
        [_jet_spec(H, f.shape[-1]) for f in feats]
        + [_jet_spec(H, p.shape[-1]) for p in pos_list]
        + [_w_spec(w.shape) for w in wc_list]
        + [_w_spec(w.shape) for w in wb_list]
        + [_w_spec(b1.shape)]
    )
    out_shape = [
        jax.ShapeDtypeStruct((NJ, H, L), _f32),
        jax.ShapeDtypeStruct((NJ, H, L), _f32),
        jax.ShapeDtypeStruct((NJ, H, K), _f32),
        jax.ShapeDtypeStruct((8, L), _f32),
    ]
    out_specs = [_jet_spec(H, L), _jet_spec(H, L), _jet_spec(H, K),
                 _w_spec((8, L))]
    return pl.pallas_call(
        body, grid=(NJ,), in_specs=in_specs, out_specs=out_specs,
        out_shape=out_shape,
    )(*feats, *pos_list, *wc_list, *wb_list, b1)


def _run_mid(a, bv, nbrf, st1, gamma, beta, w2, b2, L):
    def body(a_ref, bv_ref, nbr_ref, st_ref, g_ref, be_ref, w_ref, b2_ref,
             h2_out, st2_out):
        b = pl.program_id(0)
        s, t = _bn_consts(st_ref, g_ref, be_ref)
        rep, g = _build_g(nbr_ref[0])
        m1 = jnp.maximum(_dot(rep, a_ref[0] * s) + _dot(g, bv_ref[0] * s) + t,
                         0.0)
        h2 = _dot(m1, w_ref[...]) + b2_ref[...]
        h2_out[0] = h2
        _acc_stats(st2_out, h2, b)

    return pl.pallas_call(
        body, grid=(NJ,),
        in_specs=[_jet_spec(H, L), _jet_spec(H, L), _jet_spec(H, K),
                  _w_spec((8, L)), _w_spec((1, L)), _w_spec((1, L)),
                  _w_spec((L, L)), _w_spec((1, L))],
        out_specs=[_jet_spec(E, L), _w_spec((8, L))],
        out_shape=[jax.ShapeDtypeStruct((NJ, E, L), _f32),
                   jax.ShapeDtypeStruct((8, L), _f32)],
    )(a, bv, nbrf, st1, gamma, beta, w2, b2)


def _run_mid2(h2, st2, gamma, beta, w3, b3, L):
    def body(h_ref, st_ref, g_ref, be_ref, w_ref, b3_ref, h3_out, st3_out):
        b = pl.program_id(0)
        s, t = _bn_consts(st_ref, g_ref, be_ref)
        m2 = jnp.maximum(s * h_ref[0] + t, 0.0)
        h3 = _dot(m2, w_ref[...]) + b3_ref[...]
        h3_out[0] = h3
        _acc_stats(st3_out, h3, b)

    return pl.pallas_call(
        body, grid=(NJ,),
        in_specs=[_jet_spec(E, L), _w_spec((8, L)), _w_spec((1, L)),
                  _w_spec((1, L)), _w_spec((L, L)), _w_spec((1, L))],
        out_specs=[_jet_spec(E, L), _w_spec((8, L))],
        out_shape=[jax.ShapeDtypeStruct((NJ, E, L), _f32),
                   jax.ShapeDtypeStruct((8, L), _f32)],
    )(h2, st2, gamma, beta, w3, b3)


def _rept():
    return (_iota((H, E), 1) // K == _iota((H, E), 0)).astype(_f32)


def _run_agg(h3, st3, gamma, beta, L):
    def body(h_ref, st_ref, g_ref, be_ref, hn_out):
        s, t = _bn_consts(st_ref, g_ref, be_ref)
        m3 = jnp.maximum(s * h_ref[0] + t, 0.0)
        hn_out[0] = _dot(_rept(), m3) * (1.0 / K)

    return pl.pallas_call(
        body, grid=(NJ,),
        in_specs=[_jet_spec(E, L), _w_spec((8, L)), _w_spec((1, L)),
                  _w_spec((1, L))],
        out_specs=_jet_spec(H, L),
        out_shape=jax.ShapeDtypeStruct((NJ, H, L), _f32),
    )(h3, st3, gamma, beta)


def _run_final(h3, st3, gamma, beta, prev_feats, w1_parts, b1f, w2f, b2f, L):
    nprev = len(prev_feats)

    def body(*refs):
        i = 0
        h_ref, st_ref, g_ref, be_ref = refs[i:i + 4]; i += 4
        prefs = refs[i:i + nprev]; i += nprev
        w1refs = refs[i:i + 1 + nprev]; i += 1 + nprev
        b1_ref, w2_ref, b2_ref = refs[i:i + 3]; i += 3
        out_ref = refs[i]; i += 1
        yacc = refs[i]
        j = pl.program_id(0)

        s, t = _bn_consts(st_ref, g_ref, be_ref)
        m3 = jnp.maximum(s * h_ref[0] + t, 0.0)
        hn = _dot(_rept(), m3) * (1.0 / K)

        y = jnp.sum(hn, axis=0, keepdims=True) * (1.0 / H)
        y = _dot(y, w1refs[0][...]) + b1_ref[...]
        for pr, wr in zip(prefs, w1refs[1:]):
            pm = jnp.sum(pr[0], axis=0, keepdims=True) * (1.0 / H)
            y = y + _dot(pm, wr[...])
        yacc[pl.ds(j, 1), :] = y

        @pl.when(j == NJ - 1)
        def _():
            out_ref[...] = _dot(yacc[...], w2_ref[...]) + b2_ref[...]

    in_specs = (
        [_jet_spec(E, L), _w_spec((8, L)), _w_spec((1, L)), _w_spec((1, L))]
        + [_jet_spec(H, f.shape[-1]) for f in prev_feats]
        + [_w_spec(w.shape) for w in w1_parts]
        + [_w_spec(b1f.shape), _w_spec(w2f.shape), _w_spec(b2f.shape)]
    )
    return pl.pallas_call(
        body, grid=(NJ,),
        in_specs=in_specs,
        out_specs=pl.BlockSpec((NJ, 5), lambda b: (0, 0)),
        out_shape=jax.ShapeDtypeStruct((NJ, 5), _f32),
        scratch_shapes=[pltpu.VMEM((NJ, 256), _f32)],
    )(h3, st3, gamma, beta, *prev_feats, *w1_parts, b1f, w2f, b2f)


def _conv_weights(layers, ins, piece_dims):
    w1 = layers[0]["W"]
    wa, wb = w1[:ins], w1[ins:]
    wc = wa - wb
    offs = []
    o = 0
    for d in piece_dims:
        offs.append((o, o + d))
        o += d
    wc_list = [wc[a:b] for a, b in offs]
    wb_list = [wb[a:b] for a, b in offs]

    def row(v):
        return v.reshape(1, -1)

    return {
        "wc": wc_list, "wb": wb_list, "b1": row(layers[0]["b"]),
        "g1": row(layers[0]["gamma"]), "be1": row(layers[0]["beta"]),
        "w2": layers[1]["W"], "b2": row(layers[1]["b"]),
        "g2": row(layers[1]["gamma"]), "be2": row(layers[1]["beta"]),
        "w3": layers[2]["W"], "b3": row(layers[2]["b"]),
        "g3": row(layers[2]["gamma"]), "be3": row(layers[2]["beta"]),
    }


def _conv_block(feats, pos_list, layers, ins, L):
    cw = _conv_weights(layers, ins, [f.shape[-1] for f in feats])
    a, bv, nbrf, st1 = _run_prep(feats, pos_list, cw["wc"], cw["wb"],
                                 cw["b1"], L)
    h2, st2 = _run_mid(a, bv, nbrf, st1, cw["g1"], cw["be1"], cw["w2"],
                       cw["b2"], L)
    h3, st3 = _run_mid2(h2, st2, cw["g2"], cw["be2"], cw["w3"], cw["b3"], L)
    return h3, st3, cw


def kernel(x, params):
    x = x.astype(_f32)

    h3, st3, cw = _conv_block([x], [x[:, :, :2]], params["conv0"], 4, 64)
    h0 = _run_agg(h3, st3, cw["g3"], cw["be3"], 64)

    feats1 = [h0, x]
    h3, st3, cw = _conv_block(feats1, [], params["conv1"], 68, 128)
    h1n = _run_agg(h3, st3, cw["g3"], cw["be3"], 128)

    feats2 = [h1n, h0, x]
    h3, st3, cw = _conv_block(feats2, [], params["conv2"], 196, 256)

    w1 = params["fc1"]["W"]
    w1_parts = [w1[0:256], w1[256:384], w1[384:448], w1[448:452]]
    out = _run_final(h3, st3, cw["g3"], cw["be3"], feats2,
                     w1_parts, params["fc1"]["b"].reshape(1, -1),
                     params["fc2"]["W"], params["fc2"]["b"].reshape(1, -1),
                     256)
    return out


# speedup vs baseline: 2.6191x; 2.1982x over previous
import jax
import jax.numpy as jnp
from jax.experimental import pallas as pl
from jax.experimental.pallas import tpu as pltpu

NJ = 64
JB = 4
NS = NJ // JB
H = 100
K = 16
E = H * K
NEDGE = NJ * E
EPS = 1e-5
BIG = 1e30

_f32 = jnp.float32
_bf16 = jnp.bfloat16


def _jet_spec(*tail):
    return pl.BlockSpec((JB,) + tail, lambda b: (b, 0, 0))


def _w_spec(shape):
    return pl.BlockSpec(shape, lambda b: (0,) * len(shape))


def _iota(shape, dim, dtype=jnp.int32):
    return jax.lax.broadcasted_iota(dtype, shape, dim)


def _dotT(a, b):
    return jax.lax.dot_general(a, b, (((1,), (1,)), ((), ())),
                               preferred_element_type=_f32,
                               precision=jax.lax.Precision.HIGHEST)


def _dot(a, b):
    return jnp.dot(a, b, preferred_element_type=_f32,
                   precision=jax.lax.Precision.HIGHEST)


def _split(x):
    hi = x.astype(_bf16)
    lo = (x - hi.astype(_f32)).astype(_bf16)
    return hi, lo


def _dotb(a, b):
    return jnp.dot(a, b, preferred_element_type=_f32)


def _dot3(a, b):
    ah, al = _split(a)
    bh, bl = _split(b)
    return _dotb(ah, bh) + (_dotb(ah, bl) + _dotb(al, bh))


def _dotsel(sel_b, b):
    bh, bl = _split(b)
    return _dotb(sel_b, bh) + _dotb(sel_b, bl)


def _expand_edges(a):
    return jnp.broadcast_to(a[:, None, :], (H, K) + a.shape[1:]).reshape(
        E, a.shape[1])


def _mean_over_k(m):
    return jnp.sum(m.reshape(H, K, m.shape[1]), axis=1) * (1.0 / K)


def _build_g(nbrf):
    nbr_val = _expand_edges(nbrf)
    ksel = (_iota((E, K), 1) == _iota((E, K), 0) % K).astype(_f32)
    nbr_val = jnp.sum(nbr_val * ksel, axis=1, keepdims=True)
    return (jnp.abs(_iota((E, H), 1).astype(_f32) - nbr_val) < 0.5).astype(
        _bf16)


def _bn_consts(st_ref, g_ref, be_ref):
    mean = st_ref[0:1, :] * (1.0 / NEDGE)
    var = st_ref[1:2, :] * (1.0 / NEDGE) - mean * mean
    s = g_ref[...] * jax.lax.rsqrt(var + EPS)
    t = be_ref[...] - s * mean
    return s, t


def _init_stats(st_ref, b):
    @pl.when(b == 0)
    def _():
        st_ref[...] = jnp.zeros_like(st_ref)


def _acc_stats(st_ref, h):
    st_ref[0:1, :] = st_ref[0:1, :] + jnp.sum(h, axis=0, keepdims=True)
    st_ref[1:2, :] = st_ref[1:2, :] + jnp.sum(h * h, axis=0, keepdims=True)


def _knn_and_proj(fs, ps, wcrefs, wbrefs, b1_ref):
    d_gram = None
    for p in ps:
        d = _dotT(p, p)
        d_gram = d if d_gram is None else d_gram + d
    eye = _iota((H, H), 0) == _iota((H, H), 1)
    dm = jnp.where(eye, d_gram, 0.0)
    rdiag = jnp.sum(dm, axis=1, keepdims=True)
    cdiag = jnp.sum(dm, axis=0, keepdims=True)
    d2 = rdiag + cdiag - 2.0 * d_gram
    d2 = jnp.where(eye, BIG, d2)

    li = _iota((H, H), 1).astype(_f32)
    kl = _iota((H, K), 1)

    def step(k, carry):
        d2c, acc, adj = carry
        m = jnp.min(d2c, axis=1, keepdims=True)
        am = jnp.min(jnp.where(d2c <= m, li, 1e9), axis=1, keepdims=True)
        sel = jnp.abs(li - am) < 0.5
        d2c = jnp.where(sel, BIG, d2c)
        adj = jnp.where(sel, 1.0, adj)
        acc = jnp.where(kl == k, am, acc)
        return d2c, acc, adj

    _, nbrf, adj = jax.lax.fori_loop(
        0, K, step, (d2, jnp.zeros((H, K), _f32), jnp.zeros((H, H), _f32)))

    a = b1_ref[...]
    bv = None
    for f, wc, wb in zip(fs, wcrefs, wbrefs):
        a = a + _dot3(f, wc[...])
        pb = _dot3(f, wb[...])
        bv = pb if bv is None else bv + pb

    c_b = jnp.sum(adj, axis=0, keepdims=True).astype(_bf16)
    adj_b = adj.astype(_bf16)
    s1 = float(K) * jnp.sum(a, axis=0, keepdims=True) + _dotsel(c_b, bv)
    sq1 = (float(K) * jnp.sum(a * a, axis=0, keepdims=True)
           + _dotsel(c_b, bv * bv)
           + 2.0 * jnp.sum(a * _dotsel(adj_b, bv), axis=0, keepdims=True))
    return nbrf, a, bv, s1, sq1


def _run_prep0(feats, pos_list, wc_list, wb_list, b1, L):
    nfeat, npos = len(feats), len(pos_list)

    def body(*refs):
        i = 0
        frefs = refs[i:i + nfeat]; i += nfeat
        prefs = refs[i:i + npos]; i += npos
        wcrefs = refs[i:i + nfeat]; i += nfeat
        wbrefs = refs[i:i + nfeat]; i += nfeat
        b1_ref = refs[i]; i += 1
        a_out, bv_out, nbr_out, st_out = refs[i:i + 4]
        b = pl.program_id(0)
        _init_stats(st_out, b)
        for jj in range(JB):
            fs = [r[jj] for r in frefs]
            ps = [r[jj] for r in prefs]
            nbrf, a, bv, s1, sq1 = _knn_and_proj(fs, ps, wcrefs, wbrefs,
                                                 b1_ref)
            a_out[jj] = a
            bv_out[jj] = bv
            nbr_out[jj] = nbrf
            st_out[0:1, :] = st_out[0:1, :] + s1
            st_out[1:2, :] = st_out[1:2, :] + sq1

    in_specs = (
        [_jet_spec(H, f.shape[-1]) for f in feats]
        + [_jet_spec(H, p.shape[-1]) for p in pos_list]
        + [_w_spec(w.shape) for w in wc_list]
        + [_w_spec(w.shape) for w in wb_list]
        + [_w_spec(b1.shape)]
    )
    out_shape = [
        jax.ShapeDtypeStruct((NJ, H, L), _f32),
        jax.ShapeDtypeStruct((NJ, H, L), _f32),
        jax.ShapeDtypeStruct((NJ, H, K), _f32),
        jax.ShapeDtypeStruct((8, L), _f32),
    ]
    out_specs = [_jet_spec(H, L), _jet_spec(H, L), _jet_spec(H, K),
                 _w_spec((8, L))]
    return pl.pallas_call(
        body, grid=(NS,), in_specs=in_specs, out_specs=out_specs,
        out_shape=out_shape,
    )(*feats, *pos_list, *wc_list, *wb_list, b1)


def _run_mid(a, bv, nbrf, st1, gamma, beta, w2, b2, L):
    def body(a_ref, bv_ref, nbr_ref, st_ref, g_ref, be_ref, w_ref, b2_ref,
             h2_out, st2_out):
        b = pl.program_id(0)
        _init_stats(st2_out, b)
        s, t = _bn_consts(st_ref, g_ref, be_ref)
        for jj in range(JB):
            g = _build_g(nbr_ref[jj])
            m1 = jnp.maximum(_expand_edges(a_ref[jj] * s)
                             + _dotsel(g, bv_ref[jj] * s) + t, 0.0)
            h2 = _dot3(m1, w_ref[...]) + b2_ref[...]
            h2_out[jj] = h2
            _acc_stats(st2_out, h2)

    return pl.pallas_call(
        body, grid=(NS,),
        in_specs=[_jet_spec(H, L), _jet_spec(H, L), _jet_spec(H, K),
                  _w_spec((8, L)), _w_spec((1, L)), _w_spec((1, L)),
                  _w_spec((L, L)), _w_spec((1, L))],
        out_specs=[_jet_spec(E, L), _w_spec((8, L))],
        out_shape=[jax.ShapeDtypeStruct((NJ, E, L), _f32),
                   jax.ShapeDtypeStruct((8, L), _f32)],
    )(a, bv, nbrf, st1, gamma, beta, w2, b2)


def _run_mid2(h2, st2, gamma, beta, w3, b3, L):
    def body(h_ref, st_ref, g_ref, be_ref, w_ref, b3_ref, h3_out, st3_out):
        b = pl.program_id(0)
        _init_stats(st3_out, b)
        s, t = _bn_consts(st_ref, g_ref, be_ref)
        for jj in range(JB):
            m2 = jnp.maximum(s * h_ref[jj] + t, 0.0)
            h3 = _dot3(m2, w_ref[...]) + b3_ref[...]
            h3_out[jj] = h3
            _acc_stats(st3_out, h3)

    return pl.pallas_call(
        body, grid=(NS,),
        in_specs=[_jet_spec(E, L), _w_spec((8, L)), _w_spec((1, L)),
                  _w_spec((1, L)), _w_spec((L, L)), _w_spec((1, L))],
        out_specs=[_jet_spec(E, L), _w_spec((8, L))],
        out_shape=[jax.ShapeDtypeStruct((NJ, E, L), _f32),
                   jax.ShapeDtypeStruct((8, L), _f32)],
    )(h2, st2, gamma, beta, w3, b3)


def _run_aggprep(h3p, st3p, g3p, be3p, lp, other_feats, wc_list, wb_list,
                 b1, L):
    nof = len(other_feats)

    def body(*refs):
        i = 0
        h_ref, st_ref, g_ref, be_ref = refs[i:i + 4]; i += 4
        frefs = refs[i:i + nof]; i += nof
        wcrefs = refs[i:i + 1 + nof]; i += 1 + nof
        wbrefs = refs[i:i + 1 + nof]; i += 1 + nof
        b1_ref = refs[i]; i += 1
        hn_out, a_out, bv_out, nbr_out, st_out = refs[i:i + 5]
        b = pl.program_id(0)
        _init_stats(st_out, b)
        s, t = _bn_consts(st_ref, g_ref, be_ref)
        for jj in range(JB):
            m3 = jnp.maximum(s * h_ref[jj] + t, 0.0)
            hn = _mean_over_k(m3)
            fs = [hn] + [r[jj] for r in frefs]
            nbrf, a, bv, s1, sq1 = _knn_and_proj(fs, fs, wcrefs, wbrefs,
                                                 b1_ref)
            hn_out[jj] = hn
            a_out[jj] = a
            bv_out[jj] = bv
            nbr_out[jj] = nbrf
            st_out[0:1, :] = st_out[0:1, :] + s1
            st_out[1:2, :] = st_out[1:2, :] + sq1

    in_specs = (
        [_jet_spec(E, lp), _w_spec((8, lp)), _w_spec((1, lp)),
         _w_spec((1, lp))]
        + [_jet_spec(H, f.shape[-1]) for f in other_feats]
        + [_w_spec(w.shape) for w in wc_list]
        + [_w_spec(w.shape) for w in wb_list]
        + [_w_spec(b1.shape)]
    )
    out_shape = [
        jax.ShapeDtypeStruct((NJ, H, lp), _f32),
        jax.ShapeDtypeStruct((NJ, H, L), _f32),
        jax.ShapeDtypeStruct((NJ, H, L), _f32),
        jax.ShapeDtypeStruct((NJ, H, K), _f32),
        jax.ShapeDtypeStruct((8, L), _f32),
    ]
    out_specs = [_jet_spec(H, lp), _jet_spec(H, L), _jet_spec(H, L),
                 _jet_spec(H, K), _w_spec((8, L))]
    return pl.pallas_call(
        body, grid=(NS,), in_specs=in_specs, out_specs=out_specs,
        out_shape=out_shape,
    )(h3p, st3p, g3p, be3p, *other_feats, *wc_list, *wb_list, b1)


def _run_final(h3, st3, gamma, beta, prev_feats, w1_parts, b1f, w2f, b2f, L):
    nprev = len(prev_feats)

    def body(*refs):
        i = 0
        h_ref, st_ref, g_ref, be_ref = refs[i:i + 4]; i += 4
        prefs = refs[i:i + nprev]; i += nprev
        w1refs = refs[i:i + 1 + nprev]; i += 1 + nprev
        b1_ref, w2_ref, b2_ref = refs[i:i + 3]; i += 3
        out_ref = refs[i]; i += 1
        yacc = refs[i]
        b = pl.program_id(0)
        s, t = _bn_consts(st_ref, g_ref, be_ref)
        for jj in range(JB):
            m3 = jnp.maximum(s * h_ref[jj] + t, 0.0)
            hn = _mean_over_k(m3)
            y = jnp.sum(hn, axis=0, keepdims=True) * (1.0 / H)
            y = _dot(y, w1refs[0][...]) + b1_ref[...]
            for pr, wr in zip(prefs, w1refs[1:]):
                pm = jnp.sum(pr[jj], axis=0, keepdims=True) * (1.0 / H)
                y = y + _dot(pm, wr[...])
            yacc[pl.ds(b * JB + jj, 1), :] = y

        @pl.when(b == NS - 1)
        def _():
            out_ref[...] = _dot(yacc[...], w2_ref[...]) + b2_ref[...]

    in_specs = (
        [_jet_spec(E, L), _w_spec((8, L)), _w_spec((1, L)), _w_spec((1, L))]
        + [_jet_spec(H, f.shape[-1]) for f in prev_feats]
        + [_w_spec(w.shape) for w in w1_parts]
        + [_w_spec(b1f.shape), _w_spec(w2f.shape), _w_spec(b2f.shape)]
    )
    return pl.pallas_call(
        body, grid=(NS,),
        in_specs=in_specs,
        out_specs=pl.BlockSpec((NJ, 5), lambda b: (0, 0)),
        out_shape=jax.ShapeDtypeStruct((NJ, 5), _f32),
        scratch_shapes=[pltpu.VMEM((NJ, 256), _f32)],
    )(h3, st3, gamma, beta, *prev_feats, *w1_parts, b1f, w2f, b2f)


def _conv_weights(layers, ins, piece_dims):
    w1 = layers[0]["W"]
    wa, wb = w1[:ins], w1[ins:]
    wc = wa - wb
    offs = []
    o = 0
    for d in piece_dims:
        offs.append((o, o + d))
        o += d
    wc_list = [wc[a:b] for a, b in offs]
    wb_list = [wb[a:b] for a, b in offs]

    def row(v):
        return v.reshape(1, -1)

    return {
        "wc": wc_list, "wb": wb_list, "b1": row(layers[0]["b"]),
        "g1": row(layers[0]["gamma"]), "be1": row(layers[0]["beta"]),
        "w2": layers[1]["W"], "b2": row(layers[1]["b"]),
        "g2": row(layers[1]["gamma"]), "be2": row(layers[1]["beta"]),
        "w3": layers[2]["W"], "b3": row(layers[2]["b"]),
        "g3": row(layers[2]["gamma"]), "be3": row(layers[2]["beta"]),
    }


def kernel(x, params):
    x = x.astype(_f32)

    cw0 = _conv_weights(params["conv0"], 4, [4])
    a, bv, nbrf, st1 = _run_prep0([x], [x[:, :, :2]], cw0["wc"], cw0["wb"],
                                  cw0["b1"], 64)
    h2, st2 = _run_mid(a, bv, nbrf, st1, cw0["g1"], cw0["be1"], cw0["w2"],
                       cw0["b2"], 64)
    h3, st3 = _run_mid2(h2, st2, cw0["g2"], cw0["be2"], cw0["w3"], cw0["b3"],
                        64)

    cw1 = _conv_weights(params["conv1"], 68, [64, 4])
    h0, a, bv, nbrf, st1 = _run_aggprep(h3, st3, cw0["g3"], cw0["be3"], 64,
                                        [x], cw1["wc"], cw1["wb"], cw1["b1"],
                                        128)
    h2, st2 = _run_mid(a, bv, nbrf, st1, cw1["g1"], cw1["be1"], cw1["w2"],
                       cw1["b2"], 128)
    h3, st3 = _run_mid2(h2, st2, cw1["g2"], cw1["be2"], cw1["w3"], cw1["b3"],
                        128)

    cw2 = _conv_weights(params["conv2"], 196, [128, 64, 4])
    h1n, a, bv, nbrf, st1 = _run_aggprep(h3, st3, cw1["g3"], cw1["be3"], 128,
                                         [h0, x], cw2["wc"], cw2["wb"],
                                         cw2["b1"], 256)
    h2, st2 = _run_mid(a, bv, nbrf, st1, cw2["g1"], cw2["be1"], cw2["w2"],
                       cw2["b2"], 256)
    h3, st3 = _run_mid2(h2, st2, cw2["g2"], cw2["be2"], cw2["w3"], cw2["b3"],
                        256)

    w1 = params["fc1"]["W"]
    w1_parts = [w1[0:256], w1[256:384], w1[384:448], w1[448:452]]
    out = _run_final(h3, st3, cw2["g3"], cw2["be3"], [h1n, h0, x],
                     w1_parts, params["fc1"]["b"].reshape(1, -1),
                     params["fc2"]["W"], params["fc2"]["b"].reshape(1, -1),
                     256)
    return out


# speedup vs baseline: 2.7936x; 1.0666x over previous
import jax
import jax.numpy as jnp
from jax.experimental import pallas as pl
from jax.experimental.pallas import tpu as pltpu

NJ = 64
JB = 4
NS = NJ // JB
H = 100
K = 16
E = H * K
NEDGE = NJ * E
EPS = 1e-5
BIG = 1e30

_f32 = jnp.float32
_bf16 = jnp.bfloat16


def _jet_spec(*tail):
    return pl.BlockSpec((JB,) + tail, lambda b: (b, 0, 0))


def _w_spec(shape):
    return pl.BlockSpec(shape, lambda b: (0,) * len(shape))


def _iota(shape, dim, dtype=jnp.int32):
    return jax.lax.broadcasted_iota(dtype, shape, dim)


def _dotT(a, b):
    return jax.lax.dot_general(a, b, (((1,), (1,)), ((), ())),
                               preferred_element_type=_f32,
                               precision=jax.lax.Precision.HIGHEST)


def _dot(a, b):
    return jnp.dot(a, b, preferred_element_type=_f32,
                   precision=jax.lax.Precision.HIGHEST)


def _split(x):
    hi = x.astype(_bf16)
    lo = (x - hi.astype(_f32)).astype(_bf16)
    return hi, lo


def _dotb(a, b):
    return jnp.dot(a, b, preferred_element_type=_f32)


def _dot3(a, b):
    ah, al = _split(a)
    bh, bl = _split(b)
    return _dotb(ah, bh) + (_dotb(ah, bl) + _dotb(al, bh))


def _dot2(a, b):
    ah = a.astype(_bf16)
    bh, bl = _split(b)
    return _dotb(ah, bh) + _dotb(ah, bl)


def _dotsel(sel_b, b):
    bh, bl = _split(b)
    return _dotb(sel_b, bh) + _dotb(sel_b, bl)


def _expand_edges(a):
    return jnp.broadcast_to(a[:, None, :], (H, K) + a.shape[1:]).reshape(
        E, a.shape[1])


def _mean_over_k(m):
    return jnp.sum(m.reshape(H, K, m.shape[1]), axis=1) * (1.0 / K)


def _build_g(nbrf):
    nbr_val = _expand_edges(nbrf)
    ksel = (_iota((E, K), 1) == _iota((E, K), 0) % K).astype(_f32)
    nbr_val = jnp.sum(nbr_val * ksel, axis=1, keepdims=True)
    return (jnp.abs(_iota((E, H), 1).astype(_f32) - nbr_val) < 0.5).astype(
        _bf16)


def _bn_consts(st_ref, g_ref, be_ref):
    mean = st_ref[0:1, :] * (1.0 / NEDGE)
    var = st_ref[1:2, :] * (1.0 / NEDGE) - mean * mean
    s = g_ref[...] * jax.lax.rsqrt(var + EPS)
    t = be_ref[...] - s * mean
    return s, t


def _init_stats(st_ref, b):
    @pl.when(b == 0)
    def _():
        st_ref[...] = jnp.zeros_like(st_ref)


def _acc_stats(st_ref, h):
    st_ref[0:1, :] = st_ref[0:1, :] + jnp.sum(h, axis=0, keepdims=True)
    st_ref[1:2, :] = st_ref[1:2, :] + jnp.sum(h * h, axis=0, keepdims=True)


def _knn_and_proj(fs, ps, wcrefs, wbrefs, b1_ref, vdot=_dot3):
    d_gram = None
    for p in ps:
        d = _dotT(p, p)
        d_gram = d if d_gram is None else d_gram + d
    eye = _iota((H, H), 0) == _iota((H, H), 1)
    dm = jnp.where(eye, d_gram, 0.0)
    rdiag = jnp.sum(dm, axis=1, keepdims=True)
    cdiag = jnp.sum(dm, axis=0, keepdims=True)
    d2 = rdiag + cdiag - 2.0 * d_gram
    d2 = jnp.where(eye, BIG, d2)

    li = _iota((H, H), 1).astype(_f32)
    kl = _iota((H, K), 1)

    def step(k, carry):
        d2c, acc, adj = carry
        m = jnp.min(d2c, axis=1, keepdims=True)
        am = jnp.min(jnp.where(d2c <= m, li, 1e9), axis=1, keepdims=True)
        sel = jnp.abs(li - am) < 0.5
        d2c = jnp.where(sel, BIG, d2c)
        adj = jnp.where(sel, 1.0, adj)
        acc = jnp.where(kl == k, am, acc)
        return d2c, acc, adj

    _, nbrf, adj = jax.lax.fori_loop(
        0, K, step, (d2, jnp.zeros((H, K), _f32), jnp.zeros((H, H), _f32)))

    a = b1_ref[...]
    bv = None
    for f, wc, wb in zip(fs, wcrefs, wbrefs):
        a = a + vdot(f, wc[...])
        pb = vdot(f, wb[...])
        bv = pb if bv is None else bv + pb

    c_b = jnp.sum(adj, axis=0, keepdims=True).astype(_bf16)
    adj_b = adj.astype(_bf16)
    s1 = float(K) * jnp.sum(a, axis=0, keepdims=True) + _dotsel(c_b, bv)
    sq1 = (float(K) * jnp.sum(a * a, axis=0, keepdims=True)
           + _dotsel(c_b, bv * bv)
           + 2.0 * jnp.sum(a * _dotsel(adj_b, bv), axis=0, keepdims=True))
    return nbrf, a, bv, s1, sq1


def _run_prep0(feats, pos_list, wc_list, wb_list, b1, L):
    nfeat, npos = len(feats), len(pos_list)

    def body(*refs):
        i = 0
        frefs = refs[i:i + nfeat]; i += nfeat
        prefs = refs[i:i + npos]; i += npos
        wcrefs = refs[i:i + nfeat]; i += nfeat
        wbrefs = refs[i:i + nfeat]; i += nfeat
        b1_ref = refs[i]; i += 1
        a_out, bv_out, nbr_out, st_out = refs[i:i + 4]
        b = pl.program_id(0)
        _init_stats(st_out, b)
        for jj in range(JB):
            fs = [r[jj] for r in frefs]
            ps = [r[jj] for r in prefs]
            nbrf, a, bv, s1, sq1 = _knn_and_proj(fs, ps, wcrefs, wbrefs,
                                                 b1_ref)
            a_out[jj] = a
            bv_out[jj] = bv
            nbr_out[jj] = nbrf
            st_out[0:1, :] = st_out[0:1, :] + s1
            st_out[1:2, :] = st_out[1:2, :] + sq1

    in_specs = (
        [_jet_spec(H, f.shape[-1]) for f in feats]
        + [_jet_spec(H, p.shape[-1]) for p in pos_list]
        + [_w_spec(w.shape) for w in wc_list]
        + [_w_spec(w.shape) for w in wb_list]
        + [_w_spec(b1.shape)]
    )
    out_shape = [
        jax.ShapeDtypeStruct((NJ, H, L), _f32),
        jax.ShapeDtypeStruct((NJ, H, L), _f32),
        jax.ShapeDtypeStruct((NJ, H, K), _f32),
        jax.ShapeDtypeStruct((8, L), _f32),
    ]
    out_specs = [_jet_spec(H, L), _jet_spec(H, L), _jet_spec(H, K),
                 _w_spec((8, L))]
    return pl.pallas_call(
        body, grid=(NS,), in_specs=in_specs, out_specs=out_specs,
        out_shape=out_shape,
    )(*feats, *pos_list, *wc_list, *wb_list, b1)


def _run_mid(a, bv, nbrf, st1, gamma, beta, w2, b2, L, vdot=_dot3,
             store_dtype=_f32):
    def body(a_ref, bv_ref, nbr_ref, st_ref, g_ref, be_ref, w_ref, b2_ref,
             h2_out, st2_out):
        b = pl.program_id(0)
        _init_stats(st2_out, b)
        s, t = _bn_consts(st_ref, g_ref, be_ref)
        for jj in range(JB):
            g = _build_g(nbr_ref[jj])
            m1 = jnp.maximum(_expand_edges(a_ref[jj] * s)
                             + _dotsel(g, bv_ref[jj] * s) + t, 0.0)
            h2 = vdot(m1, w_ref[...]) + b2_ref[...]
            h2_out[jj] = h2.astype(store_dtype)
            _acc_stats(st2_out, h2)

    return pl.pallas_call(
        body, grid=(NS,),
        in_specs=[_jet_spec(H, L), _jet_spec(H, L), _jet_spec(H, K),
                  _w_spec((8, L)), _w_spec((1, L)), _w_spec((1, L)),
                  _w_spec((L, L)), _w_spec((1, L))],
        out_specs=[_jet_spec(E, L), _w_spec((8, L))],
        out_shape=[jax.ShapeDtypeStruct((NJ, E, L), store_dtype),
                   jax.ShapeDtypeStruct((8, L), _f32)],
    )(a, bv, nbrf, st1, gamma, beta, w2, b2)


def _run_mid2(h2, st2, gamma, beta, w3, b3, L, vdot=_dot3, store_dtype=_f32):
    def body(h_ref, st_ref, g_ref, be_ref, w_ref, b3_ref, h3_out, st3_out):
        b = pl.program_id(0)
        _init_stats(st3_out, b)
        s, t = _bn_consts(st_ref, g_ref, be_ref)
        for jj in range(JB):
            m2 = jnp.maximum(s * h_ref[jj].astype(_f32) + t, 0.0)
            h3 = vdot(m2, w_ref[...]) + b3_ref[...]
            h3_out[jj] = h3.astype(store_dtype)
            _acc_stats(st3_out, h3)

    return pl.pallas_call(
        body, grid=(NS,),
        in_specs=[_jet_spec(E, L), _w_spec((8, L)), _w_spec((1, L)),
                  _w_spec((1, L)), _w_spec((L, L)), _w_spec((1, L))],
        out_specs=[_jet_spec(E, L), _w_spec((8, L))],
        out_shape=[jax.ShapeDtypeStruct((NJ, E, L), store_dtype),
                   jax.ShapeDtypeStruct((8, L), _f32)],
    )(h2, st2, gamma, beta, w3, b3)


def _run_aggprep(h3p, st3p, g3p, be3p, lp, other_feats, wc_list, wb_list,
                 b1, L, vdot=_dot3):
    nof = len(other_feats)

    def body(*refs):
        i = 0
        h_ref, st_ref, g_ref, be_ref = refs[i:i + 4]; i += 4
        frefs = refs[i:i + nof]; i += nof
        wcrefs = refs[i:i + 1 + nof]; i += 1 + nof
        wbrefs = refs[i:i + 1 + nof]; i += 1 + nof
        b1_ref = refs[i]; i += 1
        hn_out, a_out, bv_out, nbr_out, st_out = refs[i:i + 5]
        b = pl.program_id(0)
        _init_stats(st_out, b)
        s, t = _bn_consts(st_ref, g_ref, be_ref)
        for jj in range(JB):
            m3 = jnp.maximum(s * h_ref[jj] + t, 0.0)
            hn = _mean_over_k(m3)
            fs = [hn] + [r[jj] for r in frefs]
            nbrf, a, bv, s1, sq1 = _knn_and_proj(fs, fs, wcrefs, wbrefs,
                                                 b1_ref, vdot)
            hn_out[jj] = hn
            a_out[jj] = a
            bv_out[jj] = bv
            nbr_out[jj] = nbrf
            st_out[0:1, :] = st_out[0:1, :] + s1
            st_out[1:2, :] = st_out[1:2, :] + sq1

    in_specs = (
        [_jet_spec(E, lp), _w_spec((8, lp)), _w_spec((1, lp)),
         _w_spec((1, lp))]
        + [_jet_spec(H, f.shape[-1]) for f in other_feats]
        + [_w_spec(w.shape) for w in wc_list]
        + [_w_spec(w.shape) for w in wb_list]
        + [_w_spec(b1.shape)]
    )
    out_shape = [
        jax.ShapeDtypeStruct((NJ, H, lp), _f32),
        jax.ShapeDtypeStruct((NJ, H, L), _f32),
        jax.ShapeDtypeStruct((NJ, H, L), _f32),
        jax.ShapeDtypeStruct((NJ, H, K), _f32),
        jax.ShapeDtypeStruct((8, L), _f32),
    ]
    out_specs = [_jet_spec(H, lp), _jet_spec(H, L), _jet_spec(H, L),
                 _jet_spec(H, K), _w_spec((8, L))]
    return pl.pallas_call(
        body, grid=(NS,), in_specs=in_specs, out_specs=out_specs,
        out_shape=out_shape,
    )(h3p, st3p, g3p, be3p, *other_feats, *wc_list, *wb_list, b1)


def _run_final(h3, st3, gamma, beta, prev_feats, w1_parts, b1f, w2f, b2f, L):
    nprev = len(prev_feats)

    def body(*refs):
        i = 0
        h_ref, st_ref, g_ref, be_ref = refs[i:i + 4]; i += 4
        prefs = refs[i:i + nprev]; i += nprev
        w1refs = refs[i:i + 1 + nprev]; i += 1 + nprev
        b1_ref, w2_ref, b2_ref = refs[i:i + 3]; i += 3
        out_ref = refs[i]; i += 1
        yacc = refs[i]
        b = pl.program_id(0)
        s, t = _bn_consts(st_ref, g_ref, be_ref)
        for jj in range(JB):
            m3 = jnp.maximum(s * h_ref[jj] + t, 0.0)
            hn = _mean_over_k(m3)
            y = jnp.sum(hn, axis=0, keepdims=True) * (1.0 / H)
            y = _dot(y, w1refs[0][...]) + b1_ref[...]
            for pr, wr in zip(prefs, w1refs[1:]):
                pm = jnp.sum(pr[jj], axis=0, keepdims=True) * (1.0 / H)
                y = y + _dot(pm, wr[...])
            yacc[pl.ds(b * JB + jj, 1), :] = y

        @pl.when(b == NS - 1)
        def _():
            out_ref[...] = _dot(yacc[...], w2_ref[...]) + b2_ref[...]

    in_specs = (
        [_jet_spec(E, L), _w_spec((8, L)), _w_spec((1, L)), _w_spec((1, L))]
        + [_jet_spec(H, f.shape[-1]) for f in prev_feats]
        + [_w_spec(w.shape) for w in w1_parts]
        + [_w_spec(b1f.shape), _w_spec(w2f.shape), _w_spec(b2f.shape)]
    )
    return pl.pallas_call(
        body, grid=(NS,),
        in_specs=in_specs,
        out_specs=pl.BlockSpec((NJ, 5), lambda b: (0, 0)),
        out_shape=jax.ShapeDtypeStruct((NJ, 5), _f32),
        scratch_shapes=[pltpu.VMEM((NJ, 256), _f32)],
    )(h3, st3, gamma, beta, *prev_feats, *w1_parts, b1f, w2f, b2f)


def _conv_weights(layers, ins, piece_dims):
    w1 = layers[0]["W"]
    wa, wb = w1[:ins], w1[ins:]
    wc = wa - wb
    offs = []
    o = 0
    for d in piece_dims:
        offs.append((o, o + d))
        o += d
    wc_list = [wc[a:b] for a, b in offs]
    wb_list = [wb[a:b] for a, b in offs]

    def row(v):
        return v.reshape(1, -1)

    return {
        "wc": wc_list, "wb": wb_list, "b1": row(layers[0]["b"]),
        "g1": row(layers[0]["gamma"]), "be1": row(layers[0]["beta"]),
        "w2": layers[1]["W"], "b2": row(layers[1]["b"]),
        "g2": row(layers[1]["gamma"]), "be2": row(layers[1]["beta"]),
        "w3": layers[2]["W"], "b3": row(layers[2]["b"]),
        "g3": row(layers[2]["gamma"]), "be3": row(layers[2]["beta"]),
    }


def kernel(x, params):
    x = x.astype(_f32)

    cw0 = _conv_weights(params["conv0"], 4, [4])
    a, bv, nbrf, st1 = _run_prep0([x], [x[:, :, :2]], cw0["wc"], cw0["wb"],
                                  cw0["b1"], 64)
    h2, st2 = _run_mid(a, bv, nbrf, st1, cw0["g1"], cw0["be1"], cw0["w2"],
                       cw0["b2"], 64)
    h3, st3 = _run_mid2(h2, st2, cw0["g2"], cw0["be2"], cw0["w3"], cw0["b3"],
                        64)

    cw1 = _conv_weights(params["conv1"], 68, [64, 4])
    h0, a, bv, nbrf, st1 = _run_aggprep(h3, st3, cw0["g3"], cw0["be3"], 64,
                                        [x], cw1["wc"], cw1["wb"], cw1["b1"],
                                        128)
    h2, st2 = _run_mid(a, bv, nbrf, st1, cw1["g1"], cw1["be1"], cw1["w2"],
                       cw1["b2"], 128)
    h3, st3 = _run_mid2(h2, st2, cw1["g2"], cw1["be2"], cw1["w3"], cw1["b3"],
                        128)

    cw2 = _conv_weights(params["conv2"], 196, [128, 64, 4])
    h1n, a, bv, nbrf, st1 = _run_aggprep(h3, st3, cw1["g3"], cw1["be3"], 128,
                                         [h0, x], cw2["wc"], cw2["wb"],
                                         cw2["b1"], 256, vdot=_dot2)
    h2, st2 = _run_mid(a, bv, nbrf, st1, cw2["g1"], cw2["be1"], cw2["w2"],
                       cw2["b2"], 256, vdot=_dot2, store_dtype=_bf16)
    h3, st3 = _run_mid2(h2, st2, cw2["g2"], cw2["be2"], cw2["w3"], cw2["b3"],
                        256, vdot=_dot2, store_dtype=_bf16)

    w1 = params["fc1"]["W"]
    w1_parts = [w1[0:256], w1[256:384], w1[384:448], w1[448:452]]
    out = _run_final(h3, st3, cw2["g3"], cw2["be3"], [h1n, h0, x],
                     w1_parts, params["fc1"]["b"].reshape(1, -1),
                     params["fc2"]["W"], params["fc2"]["b"].reshape(1, -1),
                     256)
    return out


# speedup vs baseline: 2.8290x; 1.0126x over previous
import jax
import jax.numpy as jnp
from jax.experimental import pallas as pl
from jax.experimental.pallas import tpu as pltpu

NJ = 64
JB = 8
NS = NJ // JB
H = 100
K = 16
E = H * K
NEDGE = NJ * E
EPS = 1e-5
BIG = 1e30

_f32 = jnp.float32
_bf16 = jnp.bfloat16


def _jet_spec(*tail):
    return pl.BlockSpec((JB,) + tail, lambda b: (b, 0, 0))


def _w_spec(shape):
    return pl.BlockSpec(shape, lambda b: (0,) * len(shape))


def _iota(shape, dim, dtype=jnp.int32):
    return jax.lax.broadcasted_iota(dtype, shape, dim)


def _dotT(a, b):
    return jax.lax.dot_general(a, b, (((1,), (1,)), ((), ())),
                               preferred_element_type=_f32,
                               precision=jax.lax.Precision.HIGHEST)


def _dot(a, b):
    return jnp.dot(a, b, preferred_element_type=_f32,
                   precision=jax.lax.Precision.HIGHEST)


def _split(x):
    hi = x.astype(_bf16)
    lo = (x - hi.astype(_f32)).astype(_bf16)
    return hi, lo


def _dotb(a, b):
    return jnp.dot(a, b, preferred_element_type=_f32)


def _dot3(a, b):
    ah, al = _split(a)
    bh, bl = _split(b)
    return _dotb(ah, bh) + (_dotb(ah, bl) + _dotb(al, bh))


def _dot2(a, b):
    ah = a.astype(_bf16)
    bh, bl = _split(b)
    return _dotb(ah, bh) + _dotb(ah, bl)


def _dotsel(sel_b, b):
    bh, bl = _split(b)
    return _dotb(sel_b, bh) + _dotb(sel_b, bl)


def _expand_edges(a):
    return jnp.broadcast_to(a[:, None, :], (H, K) + a.shape[1:]).reshape(
        E, a.shape[1])


def _mean_over_k(m):
    return jnp.sum(m.reshape(H, K, m.shape[1]), axis=1) * (1.0 / K)


def _build_g(nbrf):
    nbr_val = _expand_edges(nbrf)
    ksel = (_iota((E, K), 1) == _iota((E, K), 0) % K).astype(_f32)
    nbr_val = jnp.sum(nbr_val * ksel, axis=1, keepdims=True)
    return (jnp.abs(_iota((E, H), 1).astype(_f32) - nbr_val) < 0.5).astype(
        _bf16)


def _bn_consts(st_ref, g_ref, be_ref):
    mean = st_ref[0:1, :] * (1.0 / NEDGE)
    var = st_ref[1:2, :] * (1.0 / NEDGE) - mean * mean
    s = g_ref[...] * jax.lax.rsqrt(var + EPS)
    t = be_ref[...] - s * mean
    return s, t


def _init_stats(st_ref, b):
    @pl.when(b == 0)
    def _():
        st_ref[...] = jnp.zeros_like(st_ref)


def _acc_stats(st_ref, h):
    st_ref[0:1, :] = st_ref[0:1, :] + jnp.sum(h, axis=0, keepdims=True)
    st_ref[1:2, :] = st_ref[1:2, :] + jnp.sum(h * h, axis=0, keepdims=True)


def _knn_and_proj(fs, ps, wcrefs, wbrefs, b1_ref, vdot=_dot3):
    d_gram = None
    for p in ps:
        d = _dotT(p, p)
        d_gram = d if d_gram is None else d_gram + d
    eye = _iota((H, H), 0) == _iota((H, H), 1)
    dm = jnp.where(eye, d_gram, 0.0)
    rdiag = jnp.sum(dm, axis=1, keepdims=True)
    cdiag = jnp.sum(dm, axis=0, keepdims=True)
    d2 = rdiag + cdiag - 2.0 * d_gram
    d2 = jnp.where(eye, BIG, d2)

    li = _iota((H, H), 1).astype(_f32)
    kl = _iota((H, K), 1)

    def step(k, carry):
        d2c, acc, adj = carry
        m = jnp.min(d2c, axis=1, keepdims=True)
        am = jnp.min(jnp.where(d2c <= m, li, 1e9), axis=1, keepdims=True)
        sel = jnp.abs(li - am) < 0.5
        d2c = jnp.where(sel, BIG, d2c)
        adj = jnp.where(sel, 1.0, adj)
        acc = jnp.where(kl == k, am, acc)
        return d2c, acc, adj

    _, nbrf, adj = jax.lax.fori_loop(
        0, K, step, (d2, jnp.zeros((H, K), _f32), jnp.zeros((H, H), _f32)))

    a = b1_ref[...]
    bv = None
    for f, wc, wb in zip(fs, wcrefs, wbrefs):
        a = a + vdot(f, wc[...])
        pb = vdot(f, wb[...])
        bv = pb if bv is None else bv + pb

    c_b = jnp.sum(adj, axis=0, keepdims=True).astype(_bf16)
    adj_b = adj.astype(_bf16)
    s1 = float(K) * jnp.sum(a, axis=0, keepdims=True) + _dotsel(c_b, bv)
    sq1 = (float(K) * jnp.sum(a * a, axis=0, keepdims=True)
           + _dotsel(c_b, bv * bv)
           + 2.0 * jnp.sum(a * _dotsel(adj_b, bv), axis=0, keepdims=True))
    return nbrf, a, bv, s1, sq1


def _run_prep0(feats, pos_list, wc_list, wb_list, b1, L):
    nfeat, npos = len(feats), len(pos_list)

    def body(*refs):
        i = 0
        frefs = refs[i:i + nfeat]; i += nfeat
        prefs = refs[i:i + npos]; i += npos
        wcrefs = refs[i:i + nfeat]; i += nfeat
        wbrefs = refs[i:i + nfeat]; i += nfeat
        b1_ref = refs[i]; i += 1
        a_out, bv_out, nbr_out, st_out = refs[i:i + 4]
        b = pl.program_id(0)
        _init_stats(st_out, b)
        for jj in range(JB):
            fs = [r[jj] for r in frefs]
            ps = [r[jj] for r in prefs]
            nbrf, a, bv, s1, sq1 = _knn_and_proj(fs, ps, wcrefs, wbrefs,
                                                 b1_ref)
            a_out[jj] = a
            bv_out[jj] = bv
            nbr_out[jj] = nbrf
            st_out[0:1, :] = st_out[0:1, :] + s1
            st_out[1:2, :] = st_out[1:2, :] + sq1

    in_specs = (
        [_jet_spec(H, f.shape[-1]) for f in feats]
        + [_jet_spec(H, p.shape[-1]) for p in pos_list]
        + [_w_spec(w.shape) for w in wc_list]
        + [_w_spec(w.shape) for w in wb_list]
        + [_w_spec(b1.shape)]
    )
    out_shape = [
        jax.ShapeDtypeStruct((NJ, H, L), _f32),
        jax.ShapeDtypeStruct((NJ, H, L), _f32),
        jax.ShapeDtypeStruct((NJ, H, K), _f32),
        jax.ShapeDtypeStruct((8, L), _f32),
    ]
    out_specs = [_jet_spec(H, L), _jet_spec(H, L), _jet_spec(H, K),
                 _w_spec((8, L))]
    return pl.pallas_call(
        body, grid=(NS,), in_specs=in_specs, out_specs=out_specs,
        out_shape=out_shape,
    )(*feats, *pos_list, *wc_list, *wb_list, b1)


def _run_mid(a, bv, nbrf, st1, gamma, beta, w2, b2, L, vdot=_dot3,
             store_dtype=_f32):
    def body(a_ref, bv_ref, nbr_ref, st_ref, g_ref, be_ref, w_ref, b2_ref,
             h2_out, st2_out):
        b = pl.program_id(0)
        _init_stats(st2_out, b)
        s, t = _bn_consts(st_ref, g_ref, be_ref)
        for jj in range(JB):
            g = _build_g(nbr_ref[jj])
            m1 = jnp.maximum(_expand_edges(a_ref[jj] * s)
                             + _dotsel(g, bv_ref[jj] * s) + t, 0.0)
            h2 = vdot(m1, w_ref[...]) + b2_ref[...]
            h2_out[jj] = h2.astype(store_dtype)
            _acc_stats(st2_out, h2)

    return pl.pallas_call(
        body, grid=(NS,),
        in_specs=[_jet_spec(H, L), _jet_spec(H, L), _jet_spec(H, K),
                  _w_spec((8, L)), _w_spec((1, L)), _w_spec((1, L)),
                  _w_spec((L, L)), _w_spec((1, L))],
        out_specs=[_jet_spec(E, L), _w_spec((8, L))],
        out_shape=[jax.ShapeDtypeStruct((NJ, E, L), store_dtype),
                   jax.ShapeDtypeStruct((8, L), _f32)],
    )(a, bv, nbrf, st1, gamma, beta, w2, b2)


def _run_mid2(h2, st2, gamma, beta, w3, b3, L, vdot=_dot3, store_dtype=_f32):
    def body(h_ref, st_ref, g_ref, be_ref, w_ref, b3_ref, h3_out, st3_out):
        b = pl.program_id(0)
        _init_stats(st3_out, b)
        s, t = _bn_consts(st_ref, g_ref, be_ref)
        for jj in range(JB):
            m2 = jnp.maximum(s * h_ref[jj].astype(_f32) + t, 0.0)
            h3 = vdot(m2, w_ref[...]) + b3_ref[...]
            h3_out[jj] = h3.astype(store_dtype)
            _acc_stats(st3_out, h3)

    return pl.pallas_call(
        body, grid=(NS,),
        in_specs=[_jet_spec(E, L), _w_spec((8, L)), _w_spec((1, L)),
                  _w_spec((1, L)), _w_spec((L, L)), _w_spec((1, L))],
        out_specs=[_jet_spec(E, L), _w_spec((8, L))],
        out_shape=[jax.ShapeDtypeStruct((NJ, E, L), store_dtype),
                   jax.ShapeDtypeStruct((8, L), _f32)],
    )(h2, st2, gamma, beta, w3, b3)


def _run_aggprep(h3p, st3p, g3p, be3p, lp, other_feats, wc_list, wb_list,
                 b1, L, vdot=_dot3):
    nof = len(other_feats)

    def body(*refs):
        i = 0
        h_ref, st_ref, g_ref, be_ref = refs[i:i + 4]; i += 4
        frefs = refs[i:i + nof]; i += nof
        wcrefs = refs[i:i + 1 + nof]; i += 1 + nof
        wbrefs = refs[i:i + 1 + nof]; i += 1 + nof
        b1_ref = refs[i]; i += 1
        hn_out, a_out, bv_out, nbr_out, st_out = refs[i:i + 5]
        b = pl.program_id(0)
        _init_stats(st_out, b)
        s, t = _bn_consts(st_ref, g_ref, be_ref)
        for jj in range(JB):
            m3 = jnp.maximum(s * h_ref[jj] + t, 0.0)
            hn = _mean_over_k(m3)
            fs = [hn] + [r[jj] for r in frefs]
            nbrf, a, bv, s1, sq1 = _knn_and_proj(fs, fs, wcrefs, wbrefs,
                                                 b1_ref, vdot)
            hn_out[jj] = hn
            a_out[jj] = a
            bv_out[jj] = bv
            nbr_out[jj] = nbrf
            st_out[0:1, :] = st_out[0:1, :] + s1
            st_out[1:2, :] = st_out[1:2, :] + sq1

    in_specs = (
        [_jet_spec(E, lp), _w_spec((8, lp)), _w_spec((1, lp)),
         _w_spec((1, lp))]
        + [_jet_spec(H, f.shape[-1]) for f in other_feats]
        + [_w_spec(w.shape) for w in wc_list]
        + [_w_spec(w.shape) for w in wb_list]
        + [_w_spec(b1.shape)]
    )
    out_shape = [
        jax.ShapeDtypeStruct((NJ, H, lp), _f32),
        jax.ShapeDtypeStruct((NJ, H, L), _f32),
        jax.ShapeDtypeStruct((NJ, H, L), _f32),
        jax.ShapeDtypeStruct((NJ, H, K), _f32),
        jax.ShapeDtypeStruct((8, L), _f32),
    ]
    out_specs = [_jet_spec(H, lp), _jet_spec(H, L), _jet_spec(H, L),
                 _jet_spec(H, K), _w_spec((8, L))]
    return pl.pallas_call(
        body, grid=(NS,), in_specs=in_specs, out_specs=out_specs,
        out_shape=out_shape,
    )(h3p, st3p, g3p, be3p, *other_feats, *wc_list, *wb_list, b1)


def _run_final(h3, st3, gamma, beta, prev_feats, w1_parts, b1f, w2f, b2f, L):
    nprev = len(prev_feats)

    def body(*refs):
        i = 0
        h_ref, st_ref, g_ref, be_ref = refs[i:i + 4]; i += 4
        prefs = refs[i:i + nprev]; i += nprev
        w1refs = refs[i:i + 1 + nprev]; i += 1 + nprev
        b1_ref, w2_ref, b2_ref = refs[i:i + 3]; i += 3
        out_ref = refs[i]; i += 1
        yacc = refs[i]
        b = pl.program_id(0)
        s, t = _bn_consts(st_ref, g_ref, be_ref)
        for jj in range(JB):
            m3 = jnp.maximum(s * h_ref[jj] + t, 0.0)
            hn = _mean_over_k(m3)
            y = jnp.sum(hn, axis=0, keepdims=True) * (1.0 / H)
            y = _dot(y, w1refs[0][...]) + b1_ref[...]
            for pr, wr in zip(prefs, w1refs[1:]):
                pm = jnp.sum(pr[jj], axis=0, keepdims=True) * (1.0 / H)
                y = y + _dot(pm, wr[...])
            yacc[pl.ds(b * JB + jj, 1), :] = y

        @pl.when(b == NS - 1)
        def _():
            out_ref[...] = _dot(yacc[...], w2_ref[...]) + b2_ref[...]

    in_specs = (
        [_jet_spec(E, L), _w_spec((8, L)), _w_spec((1, L)), _w_spec((1, L))]
        + [_jet_spec(H, f.shape[-1]) for f in prev_feats]
        + [_w_spec(w.shape) for w in w1_parts]
        + [_w_spec(b1f.shape), _w_spec(w2f.shape), _w_spec(b2f.shape)]
    )
    return pl.pallas_call(
        body, grid=(NS,),
        in_specs=in_specs,
        out_specs=pl.BlockSpec((NJ, 5), lambda b: (0, 0)),
        out_shape=jax.ShapeDtypeStruct((NJ, 5), _f32),
        scratch_shapes=[pltpu.VMEM((NJ, 256), _f32)],
    )(h3, st3, gamma, beta, *prev_feats, *w1_parts, b1f, w2f, b2f)


def _conv_weights(layers, ins, piece_dims):
    w1 = layers[0]["W"]
    wa, wb = w1[:ins], w1[ins:]
    wc = wa - wb
    offs = []
    o = 0
    for d in piece_dims:
        offs.append((o, o + d))
        o += d
    wc_list = [wc[a:b] for a, b in offs]
    wb_list = [wb[a:b] for a, b in offs]

    def row(v):
        return v.reshape(1, -1)

    return {
        "wc": wc_list, "wb": wb_list, "b1": row(layers[0]["b"]),
        "g1": row(layers[0]["gamma"]), "be1": row(layers[0]["beta"]),
        "w2": layers[1]["W"], "b2": row(layers[1]["b"]),
        "g2": row(layers[1]["gamma"]), "be2": row(layers[1]["beta"]),
        "w3": layers[2]["W"], "b3": row(layers[2]["b"]),
        "g3": row(layers[2]["gamma"]), "be3": row(layers[2]["beta"]),
    }


def kernel(x, params):
    x = x.astype(_f32)

    cw0 = _conv_weights(params["conv0"], 4, [4])
    a, bv, nbrf, st1 = _run_prep0([x], [x[:, :, :2]], cw0["wc"], cw0["wb"],
                                  cw0["b1"], 64)
    h2, st2 = _run_mid(a, bv, nbrf, st1, cw0["g1"], cw0["be1"], cw0["w2"],
                       cw0["b2"], 64)
    h3, st3 = _run_mid2(h2, st2, cw0["g2"], cw0["be2"], cw0["w3"], cw0["b3"],
                        64)

    cw1 = _conv_weights(params["conv1"], 68, [64, 4])
    h0, a, bv, nbrf, st1 = _run_aggprep(h3, st3, cw0["g3"], cw0["be3"], 64,
                                        [x], cw1["wc"], cw1["wb"], cw1["b1"],
                                        128)
    h2, st2 = _run_mid(a, bv, nbrf, st1, cw1["g1"], cw1["be1"], cw1["w2"],
                       cw1["b2"], 128)
    h3, st3 = _run_mid2(h2, st2, cw1["g2"], cw1["be2"], cw1["w3"], cw1["b3"],
                        128)

    cw2 = _conv_weights(params["conv2"], 196, [128, 64, 4])
    h1n, a, bv, nbrf, st1 = _run_aggprep(h3, st3, cw1["g3"], cw1["be3"], 128,
                                         [h0, x], cw2["wc"], cw2["wb"],
                                         cw2["b1"], 256, vdot=_dot2)
    h2, st2 = _run_mid(a, bv, nbrf, st1, cw2["g1"], cw2["be1"], cw2["w2"],
                       cw2["b2"], 256, vdot=_dot2, store_dtype=_bf16)
    h3, st3 = _run_mid2(h2, st2, cw2["g2"], cw2["be2"], cw2["w3"], cw2["b3"],
                        256, vdot=_dot2, store_dtype=_bf16)

    w1 = params["fc1"]["W"]
    w1_parts = [w1[0:256], w1[256:384], w1[384:448], w1[448:452]]
    out = _run_final(h3, st3, cw2["g3"], cw2["be3"], [h1n, h0, x],
                     w1_parts, params["fc1"]["b"].reshape(1, -1),
                     params["fc2"]["W"], params["fc2"]["b"].reshape(1, -1),
                     256)
    return out


# speedup vs baseline: 2.8352x; 1.0022x over previous
import jax
import jax.numpy as jnp
from jax.experimental import pallas as pl
from jax.experimental.pallas import tpu as pltpu

NJ = 64
JB = 8
NS = NJ // JB
H = 100
K = 16
E = H * K
NEDGE = NJ * E
EPS = 1e-5
BIG = 1e30

_f32 = jnp.float32
_bf16 = jnp.bfloat16


def _jet_spec(*tail):
    return pl.BlockSpec((JB,) + tail, lambda b: (b, 0, 0))


def _w_spec(shape):
    return pl.BlockSpec(shape, lambda b: (0,) * len(shape))


def _iota(shape, dim, dtype=jnp.int32):
    return jax.lax.broadcasted_iota(dtype, shape, dim)


def _dotT(a, b):
    return jax.lax.dot_general(a, b, (((1,), (1,)), ((), ())),
                               preferred_element_type=_f32,
                               precision=jax.lax.Precision.HIGHEST)


def _dot(a, b):
    return jnp.dot(a, b, preferred_element_type=_f32,
                   precision=jax.lax.Precision.HIGHEST)


def _split(x):
    hi = x.astype(_bf16)
    lo = (x - hi.astype(_f32)).astype(_bf16)
    return hi, lo


def _dotb(a, b):
    return jnp.dot(a, b, preferred_element_type=_f32)


def _dot3(a, b):
    ah, al = _split(a)
    bh, bl = _split(b)
    return _dotb(ah, bh) + (_dotb(ah, bl) + _dotb(al, bh))


def _dot2(a, b):
    ah = a.astype(_bf16)
    bh, bl = _split(b)
    return _dotb(ah, bh) + _dotb(ah, bl)


def _dotsel(sel_b, b):
    bh, bl = _split(b)
    return _dotb(sel_b, bh) + _dotb(sel_b, bl)


def _expand_edges(a):
    return jnp.broadcast_to(a[:, None, :], (H, K) + a.shape[1:]).reshape(
        E, a.shape[1])


def _mean_over_k(m):
    return jnp.sum(m.reshape(H, K, m.shape[1]), axis=1) * (1.0 / K)


def _build_g(nbrf):
    nbr_val = _expand_edges(nbrf)
    ksel = (_iota((E, K), 1) == _iota((E, K), 0) % K).astype(_f32)
    nbr_val = jnp.sum(nbr_val * ksel, axis=1, keepdims=True)
    return (jnp.abs(_iota((E, H), 1).astype(_f32) - nbr_val) < 0.5).astype(
        _bf16)


def _bn_consts(st_ref, g_ref, be_ref):
    mean = st_ref[0:1, :] * (1.0 / NEDGE)
    var = st_ref[1:2, :] * (1.0 / NEDGE) - mean * mean
    s = g_ref[...] * jax.lax.rsqrt(var + EPS)
    t = be_ref[...] - s * mean
    return s, t


def _init_stats(st_ref, b):
    @pl.when(b == 0)
    def _():
        st_ref[...] = jnp.zeros_like(st_ref)


def _acc_stats(st_ref, h):
    st_ref[0:1, :] = st_ref[0:1, :] + jnp.sum(h, axis=0, keepdims=True)
    st_ref[1:2, :] = st_ref[1:2, :] + jnp.sum(h * h, axis=0, keepdims=True)


def _knn_and_proj(fs, ps, wcrefs, wbrefs, b1_ref, vdot=_dot3):
    d_gram = None
    for p in ps:
        d = _dotT(p, p)
        d_gram = d if d_gram is None else d_gram + d
    eye = _iota((H, H), 0) == _iota((H, H), 1)
    dm = jnp.where(eye, d_gram, 0.0)
    rdiag = jnp.sum(dm, axis=1, keepdims=True)
    cdiag = jnp.sum(dm, axis=0, keepdims=True)
    d2 = rdiag + cdiag - 2.0 * d_gram
    d2 = jnp.where(eye, BIG, d2)

    li = _iota((H, H), 1).astype(_f32)
    kl = _iota((H, K), 1)

    def step(k, carry):
        d2c, acc, adj = carry
        m = jnp.min(d2c, axis=1, keepdims=True)
        am = jnp.min(jnp.where(d2c <= m, li, 1e9), axis=1, keepdims=True)
        sel = jnp.abs(li - am) < 0.5
        d2c = jnp.where(sel, BIG, d2c)
        adj = jnp.where(sel, 1.0, adj)
        acc = jnp.where(kl == k, am, acc)
        return d2c, acc, adj

    _, nbrf, adj = jax.lax.fori_loop(
        0, K, step, (d2, jnp.zeros((H, K), _f32), jnp.zeros((H, H), _f32)))

    a = b1_ref[...]
    bv = None
    for f, wc, wb in zip(fs, wcrefs, wbrefs):
        a = a + vdot(f, wc[...])
        pb = vdot(f, wb[...])
        bv = pb if bv is None else bv + pb

    c_b = jnp.sum(adj, axis=0, keepdims=True).astype(_bf16)
    adj_b = adj.astype(_bf16)
    s1 = float(K) * jnp.sum(a, axis=0, keepdims=True) + _dotsel(c_b, bv)
    sq1 = (float(K) * jnp.sum(a * a, axis=0, keepdims=True)
           + _dotsel(c_b, bv * bv)
           + 2.0 * jnp.sum(a * _dotsel(adj_b, bv), axis=0, keepdims=True))
    return nbrf, a, bv, s1, sq1


def _run_prep0(feats, pos_list, wc_list, wb_list, b1, L):
    nfeat, npos = len(feats), len(pos_list)

    def body(*refs):
        i = 0
        frefs = refs[i:i + nfeat]; i += nfeat
        prefs = refs[i:i + npos]; i += npos
        wcrefs = refs[i:i + nfeat]; i += nfeat
        wbrefs = refs[i:i + nfeat]; i += nfeat
        b1_ref = refs[i]; i += 1
        a_out, bv_out, nbr_out, st_out = refs[i:i + 4]
        b = pl.program_id(0)
        _init_stats(st_out, b)
        for jj in range(JB):
            fs = [r[jj] for r in frefs]
            ps = [r[jj] for r in prefs]
            nbrf, a, bv, s1, sq1 = _knn_and_proj(fs, ps, wcrefs, wbrefs,
                                                 b1_ref)
            a_out[jj] = a
            bv_out[jj] = bv
            nbr_out[jj] = nbrf
            st_out[0:1, :] = st_out[0:1, :] + s1
            st_out[1:2, :] = st_out[1:2, :] + sq1

    in_specs = (
        [_jet_spec(H, f.shape[-1]) for f in feats]
        + [_jet_spec(H, p.shape[-1]) for p in pos_list]
        + [_w_spec(w.shape) for w in wc_list]
        + [_w_spec(w.shape) for w in wb_list]
        + [_w_spec(b1.shape)]
    )
    out_shape = [
        jax.ShapeDtypeStruct((NJ, H, L), _f32),
        jax.ShapeDtypeStruct((NJ, H, L), _f32),
        jax.ShapeDtypeStruct((NJ, H, K), _f32),
        jax.ShapeDtypeStruct((8, L), _f32),
    ]
    out_specs = [_jet_spec(H, L), _jet_spec(H, L), _jet_spec(H, K),
                 _w_spec((8, L))]
    return pl.pallas_call(
        body, grid=(NS,), in_specs=in_specs, out_specs=out_specs,
        out_shape=out_shape,
    )(*feats, *pos_list, *wc_list, *wb_list, b1)


def _run_mid(a, bv, nbrf, st1, gamma, beta, w2, b2, L, vdot=_dot3,
             store_dtype=_f32, sel1=False):
    def body(a_ref, bv_ref, nbr_ref, st_ref, g_ref, be_ref, w_ref, b2_ref,
             h2_out, st2_out):
        b = pl.program_id(0)
        _init_stats(st2_out, b)
        s, t = _bn_consts(st_ref, g_ref, be_ref)
        for jj in range(JB):
            g = _build_g(nbr_ref[jj])
            bvs = bv_ref[jj].astype(_f32) * s
            if sel1:
                nbr_term = _dotb(g, bvs.astype(_bf16))
            else:
                nbr_term = _dotsel(g, bvs)
            m1 = jnp.maximum(_expand_edges(a_ref[jj].astype(_f32) * s)
                             + nbr_term + t, 0.0)
            h2 = vdot(m1, w_ref[...]) + b2_ref[...]
            h2_out[jj] = h2.astype(store_dtype)
            _acc_stats(st2_out, h2)

    return pl.pallas_call(
        body, grid=(NS,),
        in_specs=[_jet_spec(H, L), _jet_spec(H, L), _jet_spec(H, K),
                  _w_spec((8, L)), _w_spec((1, L)), _w_spec((1, L)),
                  _w_spec((L, L)), _w_spec((1, L))],
        out_specs=[_jet_spec(E, L), _w_spec((8, L))],
        out_shape=[jax.ShapeDtypeStruct((NJ, E, L), store_dtype),
                   jax.ShapeDtypeStruct((8, L), _f32)],
    )(a, bv, nbrf, st1, gamma, beta, w2, b2)


def _run_mid2(h2, st2, gamma, beta, w3, b3, L, vdot=_dot3, store_dtype=_f32):
    def body(h_ref, st_ref, g_ref, be_ref, w_ref, b3_ref, h3_out, st3_out):
        b = pl.program_id(0)
        _init_stats(st3_out, b)
        s, t = _bn_consts(st_ref, g_ref, be_ref)
        for jj in range(JB):
            m2 = jnp.maximum(s * h_ref[jj].astype(_f32) + t, 0.0)
            h3 = vdot(m2, w_ref[...]) + b3_ref[...]
            h3_out[jj] = h3.astype(store_dtype)
            _acc_stats(st3_out, h3)

    return pl.pallas_call(
        body, grid=(NS,),
        in_specs=[_jet_spec(E, L), _w_spec((8, L)), _w_spec((1, L)),
                  _w_spec((1, L)), _w_spec((L, L)), _w_spec((1, L))],
        out_specs=[_jet_spec(E, L), _w_spec((8, L))],
        out_shape=[jax.ShapeDtypeStruct((NJ, E, L), store_dtype),
                   jax.ShapeDtypeStruct((8, L), _f32)],
    )(h2, st2, gamma, beta, w3, b3)


def _run_aggprep(h3p, st3p, g3p, be3p, lp, other_feats, wc_list, wb_list,
                 b1, L, vdot=_dot3, ab_dtype=_f32):
    nof = len(other_feats)

    def body(*refs):
        i = 0
        h_ref, st_ref, g_ref, be_ref = refs[i:i + 4]; i += 4
        frefs = refs[i:i + nof]; i += nof
        wcrefs = refs[i:i + 1 + nof]; i += 1 + nof
        wbrefs = refs[i:i + 1 + nof]; i += 1 + nof
        b1_ref = refs[i]; i += 1
        hn_out, a_out, bv_out, nbr_out, st_out = refs[i:i + 5]
        b = pl.program_id(0)
        _init_stats(st_out, b)
        s, t = _bn_consts(st_ref, g_ref, be_ref)
        for jj in range(JB):
            m3 = jnp.maximum(s * h_ref[jj] + t, 0.0)
            hn = _mean_over_k(m3)
            fs = [hn] + [r[jj] for r in frefs]
            nbrf, a, bv, s1, sq1 = _knn_and_proj(fs, fs, wcrefs, wbrefs,
                                                 b1_ref, vdot)
            hn_out[jj] = hn
            a_out[jj] = a.astype(ab_dtype)
            bv_out[jj] = bv.astype(ab_dtype)
            nbr_out[jj] = nbrf
            st_out[0:1, :] = st_out[0:1, :] + s1
            st_out[1:2, :] = st_out[1:2, :] + sq1

    in_specs = (
        [_jet_spec(E, lp), _w_spec((8, lp)), _w_spec((1, lp)),
         _w_spec((1, lp))]
        + [_jet_spec(H, f.shape[-1]) for f in other_feats]
        + [_w_spec(w.shape) for w in wc_list]
        + [_w_spec(w.shape) for w in wb_list]
        + [_w_spec(b1.shape)]
    )
    out_shape = [
        jax.ShapeDtypeStruct((NJ, H, lp), _f32),
        jax.ShapeDtypeStruct((NJ, H, L), ab_dtype),
        jax.ShapeDtypeStruct((NJ, H, L), ab_dtype),
        jax.ShapeDtypeStruct((NJ, H, K), _f32),
        jax.ShapeDtypeStruct((8, L), _f32),
    ]
    out_specs = [_jet_spec(H, lp), _jet_spec(H, L), _jet_spec(H, L),
                 _jet_spec(H, K), _w_spec((8, L))]
    return pl.pallas_call(
        body, grid=(NS,), in_specs=in_specs, out_specs=out_specs,
        out_shape=out_shape,
    )(h3p, st3p, g3p, be3p, *other_feats, *wc_list, *wb_list, b1)


def _run_final(h3, st3, gamma, beta, prev_feats, w1_parts, b1f, w2f, b2f, L):
    nprev = len(prev_feats)

    def body(*refs):
        i = 0
        h_ref, st_ref, g_ref, be_ref = refs[i:i + 4]; i += 4
        prefs = refs[i:i + nprev]; i += nprev
        w1refs = refs[i:i + 1 + nprev]; i += 1 + nprev
        b1_ref, w2_ref, b2_ref = refs[i:i + 3]; i += 3
        out_ref = refs[i]; i += 1
        yacc = refs[i]
        b = pl.program_id(0)
        s, t = _bn_consts(st_ref, g_ref, be_ref)
        for jj in range(JB):
            m3 = jnp.maximum(s * h_ref[jj] + t, 0.0)
            hn = _mean_over_k(m3)
            y = jnp.sum(hn, axis=0, keepdims=True) * (1.0 / H)
            y = _dot(y, w1refs[0][...]) + b1_ref[...]
            for pr, wr in zip(prefs, w1refs[1:]):
                pm = jnp.sum(pr[jj], axis=0, keepdims=True) * (1.0 / H)
                y = y + _dot(pm, wr[...])
            yacc[pl.ds(b * JB + jj, 1), :] = y

        @pl.when(b == NS - 1)
        def _():
            out_ref[...] = _dot(yacc[...], w2_ref[...]) + b2_ref[...]

    in_specs = (
        [_jet_spec(E, L), _w_spec((8, L)), _w_spec((1, L)), _w_spec((1, L))]
        + [_jet_spec(H, f.shape[-1]) for f in prev_feats]
        + [_w_spec(w.shape) for w in w1_parts]
        + [_w_spec(b1f.shape), _w_spec(w2f.shape), _w_spec(b2f.shape)]
    )
    return pl.pallas_call(
        body, grid=(NS,),
        in_specs=in_specs,
        out_specs=pl.BlockSpec((NJ, 5), lambda b: (0, 0)),
        out_shape=jax.ShapeDtypeStruct((NJ, 5), _f32),
        scratch_shapes=[pltpu.VMEM((NJ, 256), _f32)],
    )(h3, st3, gamma, beta, *prev_feats, *w1_parts, b1f, w2f, b2f)


def _conv_weights(layers, ins, piece_dims):
    w1 = layers[0]["W"]
    wa, wb = w1[:ins], w1[ins:]
    wc = wa - wb
    offs = []
    o = 0
    for d in piece_dims:
        offs.append((o, o + d))
        o += d
    wc_list = [wc[a:b] for a, b in offs]
    wb_list = [wb[a:b] for a, b in offs]

    def row(v):
        return v.reshape(1, -1)

    return {
        "wc": wc_list, "wb": wb_list, "b1": row(layers[0]["b"]),
        "g1": row(layers[0]["gamma"]), "be1": row(layers[0]["beta"]),
        "w2": layers[1]["W"], "b2": row(layers[1]["b"]),
        "g2": row(layers[1]["gamma"]), "be2": row(layers[1]["beta"]),
        "w3": layers[2]["W"], "b3": row(layers[2]["b"]),
        "g3": row(layers[2]["gamma"]), "be3": row(layers[2]["beta"]),
    }


def kernel(x, params):
    x = x.astype(_f32)

    cw0 = _conv_weights(params["conv0"], 4, [4])
    a, bv, nbrf, st1 = _run_prep0([x], [x[:, :, :2]], cw0["wc"], cw0["wb"],
                                  cw0["b1"], 64)
    h2, st2 = _run_mid(a, bv, nbrf, st1, cw0["g1"], cw0["be1"], cw0["w2"],
                       cw0["b2"], 64)
    h3, st3 = _run_mid2(h2, st2, cw0["g2"], cw0["be2"], cw0["w3"], cw0["b3"],
                        64)

    cw1 = _conv_weights(params["conv1"], 68, [64, 4])
    h0, a, bv, nbrf, st1 = _run_aggprep(h3, st3, cw0["g3"], cw0["be3"], 64,
                                        [x], cw1["wc"], cw1["wb"], cw1["b1"],
                                        128)
    h2, st2 = _run_mid(a, bv, nbrf, st1, cw1["g1"], cw1["be1"], cw1["w2"],
                       cw1["b2"], 128)
    h3, st3 = _run_mid2(h2, st2, cw1["g2"], cw1["be2"], cw1["w3"], cw1["b3"],
                        128)

    cw2 = _conv_weights(params["conv2"], 196, [128, 64, 4])
    h1n, a, bv, nbrf, st1 = _run_aggprep(h3, st3, cw1["g3"], cw1["be3"], 128,
                                         [h0, x], cw2["wc"], cw2["wb"],
                                         cw2["b1"], 256, vdot=_dot2,
                                         ab_dtype=_bf16)
    h2, st2 = _run_mid(a, bv, nbrf, st1, cw2["g1"], cw2["be1"], cw2["w2"],
                       cw2["b2"], 256, vdot=_dot2, store_dtype=_bf16,
                       sel1=True)
    h3, st3 = _run_mid2(h2, st2, cw2["g2"], cw2["be2"], cw2["w3"], cw2["b3"],
                        256, vdot=_dot2, store_dtype=_bf16)

    w1 = params["fc1"]["W"]
    w1_parts = [w1[0:256], w1[256:384], w1[384:448], w1[448:452]]
    out = _run_final(h3, st3, cw2["g3"], cw2["be3"], [h1n, h0, x],
                     w1_parts, params["fc1"]["b"].reshape(1, -1),
                     params["fc2"]["W"], params["fc2"]["b"].reshape(1, -1),
                     256)
    return out
